# Initial kernel scaffold; baseline (speedup 1.0000x reference)
import jax
import jax.numpy as jnp
from jax.experimental import pallas as pl


def kernel(x, edge_index, edge_attr, params):
    raise NotImplementedError("write your pallas kernel here")


# speedup vs baseline: 2.7221x; 2.7221x over previous
import functools

import jax
import jax.numpy as jnp
from jax import lax
from jax.experimental import pallas as pl
from jax.experimental.pallas import tpu as pltpu
from jax.experimental.pallas import tpu_sc as plsc

N = 10000
E = 80000
DIN = 32
H = 32
ATTR = 4
NODE_DIM = 32
FH = 64

NP = 10240
NC = 2
NS = 16
NW = NC * NS
CH = 128
EPW = 2560
NCH = EPW // CH
EP = NW * EPW
ROWS_PER_SUB = NP // NS


def _elu(v):
    return jnp.where(v > 0, v, jnp.exp(v) - 1.0)


def _mesh():
    return plsc.VectorSubcoreMesh(
        core_axis_name="c", subcore_axis_name="s", num_cores=NC, num_subcores=NS
    )


@functools.lru_cache(maxsize=None)
def _build_sc_gather():
    @functools.partial(
        pl.kernel,
        out_type=jax.ShapeDtypeStruct((EP, H), jnp.float32),
        mesh=_mesh(),
        compiler_params=pltpu.CompilerParams(use_tc_tiling_on_sc=False),
        scratch_types=[
            pltpu.VMEM((NCH, CH), jnp.int32),
            pltpu.VMEM((CH, H), jnp.float32),
            pltpu.SemaphoreType.DMA,
        ],
    )
    def k(table_hbm, src_hbm, out_hbm, idx_v, rows_v, sem):
        cid = lax.axis_index("c")
        sid = lax.axis_index("s")
        wid = sid * NC + cid
        pltpu.sync_copy(src_hbm.at[wid], idx_v)
        base = wid * EPW
        for j in range(NCH):
            pltpu.async_copy(table_hbm.at[idx_v.at[j]], rows_v, sem).wait()
            pltpu.sync_copy(rows_v, out_hbm.at[pl.ds(base + j * CH, CH)])

    return k


def _sc_gather(table, src_r):
    return _build_sc_gather()(table, src_r)


@functools.lru_cache(maxsize=None)
def _build_sc_scatter():
    @functools.partial(
        pl.kernel,
        out_type=jax.ShapeDtypeStruct((NC, NP, H), jnp.float32),
        mesh=_mesh(),
        compiler_params=pltpu.CompilerParams(use_tc_tiling_on_sc=False),
        scratch_types=[
            pltpu.VMEM((NCH, CH), jnp.int32),
            pltpu.VMEM((CH, H), jnp.float32),
            pltpu.VMEM_SHARED((NP, H), jnp.float32),
            pltpu.SemaphoreType.DMA,
        ],
    )
    def k(vals_hbm, dst_hbm, zeros_hbm, out_hbm, idx_v, rows_v, accum, sem):
        cid = lax.axis_index("c")
        sid = lax.axis_index("s")
        wid = sid * NC + cid
        pltpu.sync_copy(zeros_hbm, accum.at[pl.ds(sid * ROWS_PER_SUB, ROWS_PER_SUB)])
        plsc.subcore_barrier()
        pltpu.sync_copy(dst_hbm.at[wid], idx_v)
        base = wid * EPW
        for j in range(NCH):
            pltpu.sync_copy(vals_hbm.at[pl.ds(base + j * CH, CH)], rows_v)
            pltpu.sync_copy(rows_v, accum.at[idx_v.at[j]], add=True)
        plsc.subcore_barrier()
        sl = pl.ds(sid * ROWS_PER_SUB, ROWS_PER_SUB)
        pltpu.sync_copy(accum.at[sl], out_hbm.at[cid].at[sl])

    return k


def _sc_scatter(vals, dst_r, zeros_rows):
    return _build_sc_scatter()(vals, dst_r, zeros_rows)


@functools.lru_cache(maxsize=None)
def _build_sc_gather_scatter():
    @functools.partial(
        pl.kernel,
        out_type=jax.ShapeDtypeStruct((NC, NP, H), jnp.float32),
        mesh=_mesh(),
        compiler_params=pltpu.CompilerParams(use_tc_tiling_on_sc=False),
        scratch_types=[
            pltpu.VMEM((NCH, CH), jnp.int32),
            pltpu.VMEM((NCH, CH), jnp.int32),
            pltpu.VMEM((CH, H), jnp.float32),
            pltpu.VMEM_SHARED((NP, H), jnp.float32),
            pltpu.SemaphoreType.DMA,
        ],
    )
    def k(y_hbm, src_hbm, dst_hbm, zeros_hbm, out_hbm,
          sidx_v, didx_v, rows_v, accum, sem):
        cid = lax.axis_index("c")
        sid = lax.axis_index("s")
        wid = sid * NC + cid
        pltpu.sync_copy(zeros_hbm, accum.at[pl.ds(sid * ROWS_PER_SUB, ROWS_PER_SUB)])
        plsc.subcore_barrier()
        pltpu.sync_copy(src_hbm.at[wid], sidx_v)
        pltpu.sync_copy(dst_hbm.at[wid], didx_v)
        for j in range(NCH):
            pltpu.async_copy(y_hbm.at[sidx_v.at[j]], rows_v, sem).wait()
            pltpu.sync_copy(rows_v, accum.at[didx_v.at[j]], add=True)
        plsc.subcore_barrier()
        sl = pl.ds(sid * ROWS_PER_SUB, ROWS_PER_SUB)
        pltpu.sync_copy(accum.at[sl], out_hbm.at[cid].at[sl])

    return k


def _sc_gather_scatter(y, src_r, dst_r, zeros_rows):
    return _build_sc_gather_scatter()(y, src_r, dst_r, zeros_rows)


BE = 512


def _dot(a, b):
    return jnp.dot(a, b, preferred_element_type=jnp.float32)


def _edge_body(ea_ref, xs_ref, w1, b1, w2, b2, w3, b3, r, g, out_ref):
    h = _elu(_dot(ea_ref[...], w1[...]) + b1[...])
    h = _elu(_dot(h, w2[...]) + b2[...])
    h = _elu(_dot(h, w3[...]) + b3[...])
    xr = _dot(xs_ref[...], r[...])
    out_ref[...] = _dot(xr * h, g[...])


def _tc_edge_mlp(ea, xs, w1, b1, w2, b2, w3, b3, r, g):
    grid = (EP // BE,)

    def full(shape):
        return pl.BlockSpec(shape, lambda i: (0, 0))

    return pl.pallas_call(
        _edge_body,
        grid=grid,
        in_specs=[
            pl.BlockSpec((BE, ATTR), lambda i: (i, 0)),
            pl.BlockSpec((BE, DIN), lambda i: (i, 0)),
            full((ATTR, 256)), full((1, 256)),
            full((256, 1024)), full((1, 1024)),
            full((1024, DIN * H)), full((1, DIN * H)),
            full((DIN, DIN * H)), full((DIN * H, H)),
        ],
        out_specs=pl.BlockSpec((BE, H), lambda i: (i, 0)),
        out_shape=jax.ShapeDtypeStruct((EP, H), jnp.float32),
    )(ea, xs, w1, b1, w2, b2, w3, b3, r, g)


BN = 2048


def _combine_body(x_ref, aggp_ref, rw, rb, xc_ref, y_ref):
    agg = aggp_ref[0] + aggp_ref[1]
    xc = _dot(x_ref[...], rw[...]) + agg + rb[...]
    xc_ref[...] = xc
    y_ref[...] = _elu(xc)


def _tc_combine(x, aggp, rw, rb):
    grid = (NP // BN,)
    return pl.pallas_call(
        _combine_body,
        grid=grid,
        in_specs=[
            pl.BlockSpec((BN, DIN), lambda i: (i, 0)),
            pl.BlockSpec((NC, BN, H), lambda i: (0, i, 0)),
            pl.BlockSpec((DIN, H), lambda i: (0, 0)),
            pl.BlockSpec((1, H), lambda i: (0, 0)),
        ],
        out_specs=[
            pl.BlockSpec((BN, H), lambda i: (i, 0)),
            pl.BlockSpec((BN, H), lambda i: (i, 0)),
        ],
        out_shape=[
            jax.ShapeDtypeStruct((NP, H), jnp.float32),
            jax.ShapeDtypeStruct((NP, H), jnp.float32),
        ],
    )(x, aggp, rw, rb)


def _gin_body(y_ref, nbrp_ref, w1, b1, w2, b2, w3, b3, xc_ref, yout_ref):
    t = y_ref[...] + nbrp_ref[0] + nbrp_ref[1]
    h = _elu(_dot(t, w1[...]) + b1[...])
    h = _elu(_dot(h, w2[...]) + b2[...])
    xc = _dot(h, w3[...]) + b3[...]
    xc_ref[...] = xc
    yout_ref[...] = _elu(xc)


def _tc_gin(y, nbrp, w1, b1, w2, b2, w3, b3):
    grid = (NP // BN,)

    def wspec(shape):
        return pl.BlockSpec(shape, lambda i: (0, 0))

    return pl.pallas_call(
        _gin_body,
        grid=grid,
        in_specs=[
            pl.BlockSpec((BN, H), lambda i: (i, 0)),
            pl.BlockSpec((NC, BN, H), lambda i: (0, i, 0)),
            wspec((H, H)), wspec((1, H)),
            wspec((H, H)), wspec((1, H)),
            wspec((H, H)), wspec((1, H)),
        ],
        out_specs=[
            pl.BlockSpec((BN, H), lambda i: (i, 0)),
            pl.BlockSpec((BN, H), lambda i: (i, 0)),
        ],
        out_shape=[
            jax.ShapeDtypeStruct((NP, H), jnp.float32),
            jax.ShapeDtypeStruct((NP, H), jnp.float32),
        ],
    )(y, nbrp, w1, b1, w2, b2, w3, b3)


def _gin_fnn_body(y_ref, nbrp_ref, w1, b1, w2, b2, w3, b3,
                  xc0_ref, xc1_ref, fa, fb, fc, fb1, fw2, fb2, out_ref):
    t = y_ref[...] + nbrp_ref[0] + nbrp_ref[1]
    h = _elu(_dot(t, w1[...]) + b1[...])
    h = _elu(_dot(h, w2[...]) + b2[...])
    xc2 = _dot(h, w3[...]) + b3[...]
    hh = (_dot(xc0_ref[...], fa[...]) + _dot(xc1_ref[...], fb[...])
          + _dot(xc2, fc[...]) + fb1[...])
    hh = _elu(hh)
    out_ref[...] = _dot(hh, fw2[...]) + fb2[...]


def _tc_gin_fnn(y, nbrp, w1, b1, w2, b2, w3, b3, xc0, xc1, fa, fb, fc, fb1, fw2, fb2):
    grid = (NP // BN,)

    def wspec(shape):
        return pl.BlockSpec(shape, lambda i: (0, 0))

    def nspec():
        return pl.BlockSpec((BN, H), lambda i: (i, 0))

    return pl.pallas_call(
        _gin_fnn_body,
        grid=grid,
        in_specs=[
            nspec(),
            pl.BlockSpec((NC, BN, H), lambda i: (0, i, 0)),
            wspec((H, H)), wspec((1, H)),
            wspec((H, H)), wspec((1, H)),
            wspec((H, H)), wspec((1, H)),
            nspec(), nspec(),
            wspec((H, FH)), wspec((H, FH)), wspec((H, FH)), wspec((1, FH)),
            wspec((FH, NODE_DIM)), wspec((1, NODE_DIM)),
        ],
        out_specs=pl.BlockSpec((BN, NODE_DIM), lambda i: (i, 0)),
        out_shape=jax.ShapeDtypeStruct((NP, NODE_DIM), jnp.float32),
    )(y, nbrp, w1, b1, w2, b2, w3, b3, xc0, xc1, fa, fb, fc, fb1, fw2, fb2)


def kernel(x, edge_index, edge_attr, params):
    p = params
    f32 = jnp.float32

    x_pad = jnp.zeros((NP, DIN), f32).at[:N].set(x)
    src = jnp.zeros((EP,), jnp.int32).at[:E].set(edge_index[0])
    dst = jnp.full((EP,), N, jnp.int32).at[:E].set(edge_index[1])
    src_r = src.reshape(NW, NCH, CH)
    dst_r = dst.reshape(NW, NCH, CH)
    ea_pad = jnp.zeros((EP, ATTR), f32).at[:E].set(edge_attr)
    zeros_rows = jnp.zeros((ROWS_PER_SUB, H), f32)

    ii = jnp.arange(DIN * H)
    r_mat = (ii[None, :] // H == jnp.arange(DIN)[:, None]).astype(f32)
    g_mat = (ii[:, None] % H == jnp.arange(H)[None, :]).astype(f32)

    def row(b):
        return b.reshape(1, -1)

    w1r = p['fnn_w1'].reshape(H, 3, FH)
    fa, fb, fc = w1r[:, 0, :], w1r[:, 1, :], w1r[:, 2, :]

    xs = _sc_gather(x_pad, src_r)
    msg = _tc_edge_mlp(ea_pad, xs,
                       p['e_w1'], row(p['e_b1']),
                       p['e_w2'], row(p['e_b2']),
                       p['e_w3'], row(p['e_b3']),
                       r_mat, g_mat)
    aggp = _sc_scatter(msg, dst_r, zeros_rows)
    xc0, y0 = _tc_combine(x_pad, aggp, p['root_w'], row(p['root_b']))
    nbr1p = _sc_gather_scatter(y0, src_r, dst_r, zeros_rows)
    xc1, y1 = _tc_gin(y0, nbr1p,
                      p['gin1_w1'], row(p['gin1_b1']),
                      p['gin1_w2'], row(p['gin1_b2']),
                      p['gin1_w3'], row(p['gin1_b3']))
    nbr2p = _sc_gather_scatter(y1, src_r, dst_r, zeros_rows)
    out = _tc_gin_fnn(y1, nbr2p,
                      p['gin2_w1'], row(p['gin2_b1']),
                      p['gin2_w2'], row(p['gin2_b2']),
                      p['gin2_w3'], row(p['gin2_b3']),
                      xc0, xc1, fa, fb, fc,
                      row(p['fnn_b1']), p['fnn_w2'], row(p['fnn_b2']))
    return out[:N]


# speedup vs baseline: 2.8484x; 1.0464x over previous
import functools

import jax
import jax.numpy as jnp
from jax import lax
from jax.experimental import pallas as pl
from jax.experimental.pallas import tpu as pltpu
from jax.experimental.pallas import tpu_sc as plsc

N = 10000
E = 80000
DIN = 32
H = 32
ATTR = 4
NODE_DIM = 32
FH = 64

NP = 10240
NC = 2
NS = 16
NW = NC * NS
CH = 128
EPW = 2560
NCH = EPW // CH
EP = NW * EPW
ROWS_PER_SUB = NP // NS


def _elu(v):
    return jnp.where(v > 0, v, jnp.exp(v) - 1.0)


def _mesh():
    return plsc.VectorSubcoreMesh(
        core_axis_name="c", subcore_axis_name="s", num_cores=NC, num_subcores=NS
    )


@functools.lru_cache(maxsize=None)
def _build_sc_gather():
    @functools.partial(
        pl.kernel,
        out_type=jax.ShapeDtypeStruct((EP, H), jnp.float32),
        mesh=_mesh(),
        compiler_params=pltpu.CompilerParams(use_tc_tiling_on_sc=False),
        scratch_types=[
            pltpu.VMEM((NCH, CH), jnp.int32),
            pltpu.VMEM((EPW, H), jnp.float32),
            pltpu.SemaphoreType.DMA,
        ],
    )
    def k(table_hbm, src_hbm, out_hbm, idx_v, rows_v, sem):
        cid = lax.axis_index("c")
        sid = lax.axis_index("s")
        wid = sid * NC + cid
        pltpu.sync_copy(src_hbm.at[wid], idx_v)
        cps = [
            pltpu.async_copy(
                table_hbm.at[idx_v.at[j]], rows_v.at[pl.ds(j * CH, CH)], sem
            )
            for j in range(NCH)
        ]
        for c in cps:
            c.wait()
        pltpu.sync_copy(rows_v, out_hbm.at[pl.ds(wid * EPW, EPW)])

    return k


def _sc_gather(table, src_r):
    return _build_sc_gather()(table, src_r)


@functools.lru_cache(maxsize=None)
def _build_sc_scatter():
    @functools.partial(
        pl.kernel,
        out_type=jax.ShapeDtypeStruct((NC, NP, H), jnp.float32),
        mesh=_mesh(),
        compiler_params=pltpu.CompilerParams(use_tc_tiling_on_sc=False),
        scratch_types=[
            pltpu.VMEM((NCH, CH), jnp.int32),
            pltpu.VMEM((EPW, H), jnp.float32),
            pltpu.VMEM_SHARED((NP, H), jnp.float32),
            pltpu.SemaphoreType.DMA,
        ],
    )
    def k(vals_hbm, dst_hbm, zeros_hbm, out_hbm, idx_v, rows_v, accum, sem):
        cid = lax.axis_index("c")
        sid = lax.axis_index("s")
        wid = sid * NC + cid
        pltpu.sync_copy(zeros_hbm, accum.at[pl.ds(sid * ROWS_PER_SUB, ROWS_PER_SUB)])
        pltpu.sync_copy(dst_hbm.at[wid], idx_v)
        pltpu.sync_copy(vals_hbm.at[pl.ds(wid * EPW, EPW)], rows_v)
        plsc.subcore_barrier()
        cps = [
            pltpu.async_copy(
                rows_v.at[pl.ds(j * CH, CH)], accum.at[idx_v.at[j]], sem, add=True
            )
            for j in range(NCH)
        ]
        for c in cps:
            c.wait()
        plsc.subcore_barrier()
        sl = pl.ds(sid * ROWS_PER_SUB, ROWS_PER_SUB)
        pltpu.sync_copy(accum.at[sl], out_hbm.at[cid].at[sl])

    return k


def _sc_scatter(vals, dst_r, zeros_rows):
    return _build_sc_scatter()(vals, dst_r, zeros_rows)


@functools.lru_cache(maxsize=None)
def _build_sc_gather_scatter():
    @functools.partial(
        pl.kernel,
        out_type=jax.ShapeDtypeStruct((NC, NP, H), jnp.float32),
        mesh=_mesh(),
        compiler_params=pltpu.CompilerParams(use_tc_tiling_on_sc=False),
        scratch_types=[
            pltpu.VMEM((NCH, CH), jnp.int32),
            pltpu.VMEM((NCH, CH), jnp.int32),
            pltpu.VMEM((EPW, H), jnp.float32),
            pltpu.VMEM_SHARED((NP, H), jnp.float32),
            pltpu.SemaphoreType.DMA,
            pltpu.SemaphoreType.DMA,
        ],
    )
    def k(y_hbm, src_hbm, dst_hbm, zeros_hbm, out_hbm,
          sidx_v, didx_v, rows_v, accum, gsem, ssem):
        cid = lax.axis_index("c")
        sid = lax.axis_index("s")
        wid = sid * NC + cid
        pltpu.sync_copy(zeros_hbm, accum.at[pl.ds(sid * ROWS_PER_SUB, ROWS_PER_SUB)])
        pltpu.sync_copy(src_hbm.at[wid], sidx_v)
        pltpu.sync_copy(dst_hbm.at[wid], didx_v)
        gcps = [
            pltpu.async_copy(
                y_hbm.at[sidx_v.at[j]], rows_v.at[pl.ds(j * CH, CH)], gsem
            )
            for j in range(NCH)
        ]
        plsc.subcore_barrier()
        for c in gcps:
            c.wait()
        scps = [
            pltpu.async_copy(
                rows_v.at[pl.ds(j * CH, CH)], accum.at[didx_v.at[j]], ssem, add=True
            )
            for j in range(NCH)
        ]
        for c in scps:
            c.wait()
        plsc.subcore_barrier()
        sl = pl.ds(sid * ROWS_PER_SUB, ROWS_PER_SUB)
        pltpu.sync_copy(accum.at[sl], out_hbm.at[cid].at[sl])

    return k


def _sc_gather_scatter(y, src_r, dst_r, zeros_rows):
    return _build_sc_gather_scatter()(y, src_r, dst_r, zeros_rows)


BE = 512


def _dot(a, b):
    return jnp.dot(a, b, preferred_element_type=jnp.float32)


def _edge_body(ea_ref, xs_ref, w1, b1, w2, b2, w3, b3, r, g, out_ref):
    h = _elu(_dot(ea_ref[...], w1[...]) + b1[...])
    h = _elu(_dot(h, w2[...]) + b2[...])
    h = _elu(_dot(h, w3[...]) + b3[...])
    xr = _dot(xs_ref[...], r[...])
    out_ref[...] = _dot(xr * h, g[...])


def _tc_edge_mlp(ea, xs, w1, b1, w2, b2, w3, b3, r, g):
    grid = (EP // BE,)

    def full(shape):
        return pl.BlockSpec(shape, lambda i: (0, 0))

    return pl.pallas_call(
        _edge_body,
        grid=grid,
        in_specs=[
            pl.BlockSpec((BE, ATTR), lambda i: (i, 0)),
            pl.BlockSpec((BE, DIN), lambda i: (i, 0)),
            full((ATTR, 256)), full((1, 256)),
            full((256, 1024)), full((1, 1024)),
            full((1024, DIN * H)), full((1, DIN * H)),
            full((DIN, DIN * H)), full((DIN * H, H)),
        ],
        out_specs=pl.BlockSpec((BE, H), lambda i: (i, 0)),
        out_shape=jax.ShapeDtypeStruct((EP, H), jnp.float32),
    )(ea, xs, w1, b1, w2, b2, w3, b3, r, g)


BN = 2048


def _combine_body(x_ref, aggp_ref, rw, rb, xc_ref, y_ref):
    agg = aggp_ref[0] + aggp_ref[1]
    xc = _dot(x_ref[...], rw[...]) + agg + rb[...]
    xc_ref[...] = xc
    y_ref[...] = _elu(xc)


def _tc_combine(x, aggp, rw, rb):
    grid = (NP // BN,)
    return pl.pallas_call(
        _combine_body,
        grid=grid,
        in_specs=[
            pl.BlockSpec((BN, DIN), lambda i: (i, 0)),
            pl.BlockSpec((NC, BN, H), lambda i: (0, i, 0)),
            pl.BlockSpec((DIN, H), lambda i: (0, 0)),
            pl.BlockSpec((1, H), lambda i: (0, 0)),
        ],
        out_specs=[
            pl.BlockSpec((BN, H), lambda i: (i, 0)),
            pl.BlockSpec((BN, H), lambda i: (i, 0)),
        ],
        out_shape=[
            jax.ShapeDtypeStruct((NP, H), jnp.float32),
            jax.ShapeDtypeStruct((NP, H), jnp.float32),
        ],
    )(x, aggp, rw, rb)


def _gin_body(y_ref, nbrp_ref, w1, b1, w2, b2, w3, b3, xc_ref, yout_ref):
    t = y_ref[...] + nbrp_ref[0] + nbrp_ref[1]
    h = _elu(_dot(t, w1[...]) + b1[...])
    h = _elu(_dot(h, w2[...]) + b2[...])
    xc = _dot(h, w3[...]) + b3[...]
    xc_ref[...] = xc
    yout_ref[...] = _elu(xc)


def _tc_gin(y, nbrp, w1, b1, w2, b2, w3, b3):
    grid = (NP // BN,)

    def wspec(shape):
        return pl.BlockSpec(shape, lambda i: (0, 0))

    return pl.pallas_call(
        _gin_body,
        grid=grid,
        in_specs=[
            pl.BlockSpec((BN, H), lambda i: (i, 0)),
            pl.BlockSpec((NC, BN, H), lambda i: (0, i, 0)),
            wspec((H, H)), wspec((1, H)),
            wspec((H, H)), wspec((1, H)),
            wspec((H, H)), wspec((1, H)),
        ],
        out_specs=[
            pl.BlockSpec((BN, H), lambda i: (i, 0)),
            pl.BlockSpec((BN, H), lambda i: (i, 0)),
        ],
        out_shape=[
            jax.ShapeDtypeStruct((NP, H), jnp.float32),
            jax.ShapeDtypeStruct((NP, H), jnp.float32),
        ],
    )(y, nbrp, w1, b1, w2, b2, w3, b3)


def _gin_fnn_body(y_ref, nbrp_ref, w1, b1, w2, b2, w3, b3,
                  xc0_ref, xc1_ref, fa, fb, fc, fb1, fw2, fb2, out_ref):
    t = y_ref[...] + nbrp_ref[0] + nbrp_ref[1]
    h = _elu(_dot(t, w1[...]) + b1[...])
    h = _elu(_dot(h, w2[...]) + b2[...])
    xc2 = _dot(h, w3[...]) + b3[...]
    hh = (_dot(xc0_ref[...], fa[...]) + _dot(xc1_ref[...], fb[...])
          + _dot(xc2, fc[...]) + fb1[...])
    hh = _elu(hh)
    out_ref[...] = _dot(hh, fw2[...]) + fb2[...]


def _tc_gin_fnn(y, nbrp, w1, b1, w2, b2, w3, b3, xc0, xc1, fa, fb, fc, fb1, fw2, fb2):
    grid = (NP // BN,)

    def wspec(shape):
        return pl.BlockSpec(shape, lambda i: (0, 0))

    def nspec():
        return pl.BlockSpec((BN, H), lambda i: (i, 0))

    return pl.pallas_call(
        _gin_fnn_body,
        grid=grid,
        in_specs=[
            nspec(),
            pl.BlockSpec((NC, BN, H), lambda i: (0, i, 0)),
            wspec((H, H)), wspec((1, H)),
            wspec((H, H)), wspec((1, H)),
            wspec((H, H)), wspec((1, H)),
            nspec(), nspec(),
            wspec((H, FH)), wspec((H, FH)), wspec((H, FH)), wspec((1, FH)),
            wspec((FH, NODE_DIM)), wspec((1, NODE_DIM)),
        ],
        out_specs=pl.BlockSpec((BN, NODE_DIM), lambda i: (i, 0)),
        out_shape=jax.ShapeDtypeStruct((NP, NODE_DIM), jnp.float32),
    )(y, nbrp, w1, b1, w2, b2, w3, b3, xc0, xc1, fa, fb, fc, fb1, fw2, fb2)


def kernel(x, edge_index, edge_attr, params):
    p = params
    f32 = jnp.float32

    x_pad = jnp.zeros((NP, DIN), f32).at[:N].set(x)
    src = jnp.zeros((EP,), jnp.int32).at[:E].set(edge_index[0])
    dst = jnp.full((EP,), N, jnp.int32).at[:E].set(edge_index[1])
    src_r = src.reshape(NW, NCH, CH)
    dst_r = dst.reshape(NW, NCH, CH)
    ea_pad = jnp.zeros((EP, ATTR), f32).at[:E].set(edge_attr)
    zeros_rows = jnp.zeros((ROWS_PER_SUB, H), f32)

    ii = jnp.arange(DIN * H)
    r_mat = (ii[None, :] // H == jnp.arange(DIN)[:, None]).astype(f32)
    g_mat = (ii[:, None] % H == jnp.arange(H)[None, :]).astype(f32)

    def row(b):
        return b.reshape(1, -1)

    w1r = p['fnn_w1'].reshape(H, 3, FH)
    fa, fb, fc = w1r[:, 0, :], w1r[:, 1, :], w1r[:, 2, :]

    xs = _sc_gather(x_pad, src_r)
    msg = _tc_edge_mlp(ea_pad, xs,
                       p['e_w1'], row(p['e_b1']),
                       p['e_w2'], row(p['e_b2']),
                       p['e_w3'], row(p['e_b3']),
                       r_mat, g_mat)
    aggp = _sc_scatter(msg, dst_r, zeros_rows)
    xc0, y0 = _tc_combine(x_pad, aggp, p['root_w'], row(p['root_b']))
    nbr1p = _sc_gather_scatter(y0, src_r, dst_r, zeros_rows)
    xc1, y1 = _tc_gin(y0, nbr1p,
                      p['gin1_w1'], row(p['gin1_b1']),
                      p['gin1_w2'], row(p['gin1_b2']),
                      p['gin1_w3'], row(p['gin1_b3']))
    nbr2p = _sc_gather_scatter(y1, src_r, dst_r, zeros_rows)
    out = _tc_gin_fnn(y1, nbr2p,
                      p['gin2_w1'], row(p['gin2_b1']),
                      p['gin2_w2'], row(p['gin2_b2']),
                      p['gin2_w3'], row(p['gin2_b3']),
                      xc0, xc1, fa, fb, fc,
                      row(p['fnn_b1']), p['fnn_w2'], row(p['fnn_b2']))
    return out[:N]


# speedup vs baseline: 3.0017x; 1.0538x over previous
import functools

import jax
import jax.numpy as jnp
from jax import lax
from jax.experimental import pallas as pl
from jax.experimental.pallas import tpu as pltpu
from jax.experimental.pallas import tpu_sc as plsc

N = 10000
E = 80000
DIN = 32
H = 32
ATTR = 4
NODE_DIM = 32
FH = 64

NP = 10240
NC = 2
NS = 16
NW = NC * NS
CH = 128
EPW = 2560
NCH = EPW // CH
EP = NW * EPW
ROWS_PER_SUB = NP // NS


def _elu(v):
    return jnp.where(v > 0, v, jnp.exp(v) - 1.0)


def _mesh():
    return plsc.VectorSubcoreMesh(
        core_axis_name="c", subcore_axis_name="s", num_cores=NC, num_subcores=NS
    )


@functools.lru_cache(maxsize=None)
def _build_sc_gather():
    @functools.partial(
        pl.kernel,
        out_type=jax.ShapeDtypeStruct((EP, H), jnp.float32),
        mesh=_mesh(),
        compiler_params=pltpu.CompilerParams(use_tc_tiling_on_sc=False),
        scratch_types=[
            pltpu.VMEM((NCH, CH), jnp.int32),
            pltpu.VMEM((EPW, H), jnp.float32),
            pltpu.SemaphoreType.DMA,
        ],
    )
    def k(table_hbm, src_hbm, out_hbm, idx_v, rows_v, sem):
        cid = lax.axis_index("c")
        sid = lax.axis_index("s")
        wid = sid * NC + cid
        pltpu.sync_copy(src_hbm.at[wid], idx_v)
        cps = [
            pltpu.async_copy(
                table_hbm.at[idx_v.at[j]], rows_v.at[pl.ds(j * CH, CH)], sem
            )
            for j in range(NCH)
        ]
        for c in cps:
            c.wait()
        pltpu.sync_copy(rows_v, out_hbm.at[pl.ds(wid * EPW, EPW)])

    return k


def _sc_gather(table, src_r):
    return _build_sc_gather()(table, src_r)


@functools.lru_cache(maxsize=None)
def _build_sc_scatter():
    @functools.partial(
        pl.kernel,
        out_type=jax.ShapeDtypeStruct((NC, NP, H), jnp.float32),
        mesh=_mesh(),
        compiler_params=pltpu.CompilerParams(use_tc_tiling_on_sc=False),
        scratch_types=[
            pltpu.VMEM((NCH, CH), jnp.int32),
            pltpu.VMEM((EPW, H), jnp.float32),
            pltpu.VMEM_SHARED((NP, H), jnp.float32),
            pltpu.SemaphoreType.DMA,
        ],
    )
    def k(vals_hbm, dst_hbm, zeros_hbm, out_hbm, idx_v, rows_v, accum, sem):
        cid = lax.axis_index("c")
        sid = lax.axis_index("s")
        wid = sid * NC + cid
        pltpu.sync_copy(zeros_hbm, accum.at[pl.ds(sid * ROWS_PER_SUB, ROWS_PER_SUB)])
        pltpu.sync_copy(dst_hbm.at[wid], idx_v)
        pltpu.sync_copy(vals_hbm.at[pl.ds(wid * EPW, EPW)], rows_v)
        plsc.subcore_barrier()
        cps = [
            pltpu.async_copy(
                rows_v.at[pl.ds(j * CH, CH)], accum.at[idx_v.at[j]], sem, add=True
            )
            for j in range(NCH)
        ]
        for c in cps:
            c.wait()
        plsc.subcore_barrier()
        sl = pl.ds(sid * ROWS_PER_SUB, ROWS_PER_SUB)
        pltpu.sync_copy(accum.at[sl], out_hbm.at[cid].at[sl])

    return k


def _sc_scatter(vals, dst_r, zeros_rows):
    return _build_sc_scatter()(vals, dst_r, zeros_rows)


@functools.lru_cache(maxsize=None)
def _build_sc_gather_scatter():
    @functools.partial(
        pl.kernel,
        out_type=jax.ShapeDtypeStruct((NC, NP, H), jnp.float32),
        mesh=_mesh(),
        compiler_params=pltpu.CompilerParams(use_tc_tiling_on_sc=False),
        scratch_types=[
            pltpu.VMEM((NCH, CH), jnp.int32),
            pltpu.VMEM((NCH, CH), jnp.int32),
            pltpu.VMEM((EPW, H), jnp.float32),
            pltpu.VMEM_SHARED((NP, H), jnp.float32),
            pltpu.SemaphoreType.DMA,
            pltpu.SemaphoreType.DMA,
        ],
    )
    def k(y_hbm, src_hbm, dst_hbm, zeros_hbm, out_hbm,
          sidx_v, didx_v, rows_v, accum, gsem, ssem):
        cid = lax.axis_index("c")
        sid = lax.axis_index("s")
        wid = sid * NC + cid
        pltpu.sync_copy(zeros_hbm, accum.at[pl.ds(sid * ROWS_PER_SUB, ROWS_PER_SUB)])
        pltpu.sync_copy(src_hbm.at[wid], sidx_v)
        pltpu.sync_copy(dst_hbm.at[wid], didx_v)
        gcps = [
            pltpu.async_copy(
                y_hbm.at[sidx_v.at[j]], rows_v.at[pl.ds(j * CH, CH)], gsem
            )
            for j in range(NCH)
        ]
        plsc.subcore_barrier()
        for c in gcps:
            c.wait()
        scps = [
            pltpu.async_copy(
                rows_v.at[pl.ds(j * CH, CH)], accum.at[didx_v.at[j]], ssem, add=True
            )
            for j in range(NCH)
        ]
        for c in scps:
            c.wait()
        plsc.subcore_barrier()
        sl = pl.ds(sid * ROWS_PER_SUB, ROWS_PER_SUB)
        pltpu.sync_copy(accum.at[sl], out_hbm.at[cid].at[sl])

    return k


def _sc_gather_scatter(y, src_r, dst_r, zeros_rows):
    return _build_sc_gather_scatter()(y, src_r, dst_r, zeros_rows)


BE = 2048


def _dot(a, b):
    return jnp.dot(a, b, preferred_element_type=jnp.float32)


def _edge_body(ea_ref, xs_ref, w1, b1, w2, b2, w3, b3, r, g, out_ref):
    h = _elu(_dot(ea_ref[...], w1[...]) + b1[...])
    h = _elu(_dot(h, w2[...]) + b2[...])
    h = _elu(_dot(h, w3[...]) + b3[...])
    xr = _dot(xs_ref[...], r[...])
    out_ref[...] = _dot(xr * h, g[...])


def _tc_edge_mlp(ea, xs, w1, b1, w2, b2, w3, b3, r, g):
    grid = (EP // BE,)

    def full(shape):
        return pl.BlockSpec(shape, lambda i: (0, 0))

    return pl.pallas_call(
        _edge_body,
        grid=grid,
        in_specs=[
            pl.BlockSpec((BE, ATTR), lambda i: (i, 0)),
            pl.BlockSpec((BE, DIN), lambda i: (i, 0)),
            full((ATTR, 256)), full((1, 256)),
            full((256, 1024)), full((1, 1024)),
            full((1024, DIN * H)), full((1, DIN * H)),
            full((DIN, DIN * H)), full((DIN * H, H)),
        ],
        out_specs=pl.BlockSpec((BE, H), lambda i: (i, 0)),
        out_shape=jax.ShapeDtypeStruct((EP, H), jnp.float32),
    )(ea, xs, w1, b1, w2, b2, w3, b3, r, g)


BN = 2048


def _combine_body(x_ref, aggp_ref, rw, rb, xc_ref, y_ref):
    agg = aggp_ref[0] + aggp_ref[1]
    xc = _dot(x_ref[...], rw[...]) + agg + rb[...]
    xc_ref[...] = xc
    y_ref[...] = _elu(xc)


def _tc_combine(x, aggp, rw, rb):
    grid = (NP // BN,)
    return pl.pallas_call(
        _combine_body,
        grid=grid,
        in_specs=[
            pl.BlockSpec((BN, DIN), lambda i: (i, 0)),
            pl.BlockSpec((NC, BN, H), lambda i: (0, i, 0)),
            pl.BlockSpec((DIN, H), lambda i: (0, 0)),
            pl.BlockSpec((1, H), lambda i: (0, 0)),
        ],
        out_specs=[
            pl.BlockSpec((BN, H), lambda i: (i, 0)),
            pl.BlockSpec((BN, H), lambda i: (i, 0)),
        ],
        out_shape=[
            jax.ShapeDtypeStruct((NP, H), jnp.float32),
            jax.ShapeDtypeStruct((NP, H), jnp.float32),
        ],
    )(x, aggp, rw, rb)


def _gin_body(y_ref, nbrp_ref, w1, b1, w2, b2, w3, b3, xc_ref, yout_ref):
    t = y_ref[...] + nbrp_ref[0] + nbrp_ref[1]
    h = _elu(_dot(t, w1[...]) + b1[...])
    h = _elu(_dot(h, w2[...]) + b2[...])
    xc = _dot(h, w3[...]) + b3[...]
    xc_ref[...] = xc
    yout_ref[...] = _elu(xc)


def _tc_gin(y, nbrp, w1, b1, w2, b2, w3, b3):
    grid = (NP // BN,)

    def wspec(shape):
        return pl.BlockSpec(shape, lambda i: (0, 0))

    return pl.pallas_call(
        _gin_body,
        grid=grid,
        in_specs=[
            pl.BlockSpec((BN, H), lambda i: (i, 0)),
            pl.BlockSpec((NC, BN, H), lambda i: (0, i, 0)),
            wspec((H, H)), wspec((1, H)),
            wspec((H, H)), wspec((1, H)),
            wspec((H, H)), wspec((1, H)),
        ],
        out_specs=[
            pl.BlockSpec((BN, H), lambda i: (i, 0)),
            pl.BlockSpec((BN, H), lambda i: (i, 0)),
        ],
        out_shape=[
            jax.ShapeDtypeStruct((NP, H), jnp.float32),
            jax.ShapeDtypeStruct((NP, H), jnp.float32),
        ],
    )(y, nbrp, w1, b1, w2, b2, w3, b3)


def _gin_fnn_body(y_ref, nbrp_ref, w1, b1, w2, b2, w3, b3,
                  xc0_ref, xc1_ref, fa, fb, fc, fb1, fw2, fb2, out_ref):
    t = y_ref[...] + nbrp_ref[0] + nbrp_ref[1]
    h = _elu(_dot(t, w1[...]) + b1[...])
    h = _elu(_dot(h, w2[...]) + b2[...])
    xc2 = _dot(h, w3[...]) + b3[...]
    hh = (_dot(xc0_ref[...], fa[...]) + _dot(xc1_ref[...], fb[...])
          + _dot(xc2, fc[...]) + fb1[...])
    hh = _elu(hh)
    out_ref[...] = _dot(hh, fw2[...]) + fb2[...]


def _tc_gin_fnn(y, nbrp, w1, b1, w2, b2, w3, b3, xc0, xc1, fa, fb, fc, fb1, fw2, fb2):
    grid = (NP // BN,)

    def wspec(shape):
        return pl.BlockSpec(shape, lambda i: (0, 0))

    def nspec():
        return pl.BlockSpec((BN, H), lambda i: (i, 0))

    return pl.pallas_call(
        _gin_fnn_body,
        grid=grid,
        in_specs=[
            nspec(),
            pl.BlockSpec((NC, BN, H), lambda i: (0, i, 0)),
            wspec((H, H)), wspec((1, H)),
            wspec((H, H)), wspec((1, H)),
            wspec((H, H)), wspec((1, H)),
            nspec(), nspec(),
            wspec((H, FH)), wspec((H, FH)), wspec((H, FH)), wspec((1, FH)),
            wspec((FH, NODE_DIM)), wspec((1, NODE_DIM)),
        ],
        out_specs=pl.BlockSpec((BN, NODE_DIM), lambda i: (i, 0)),
        out_shape=jax.ShapeDtypeStruct((NP, NODE_DIM), jnp.float32),
    )(y, nbrp, w1, b1, w2, b2, w3, b3, xc0, xc1, fa, fb, fc, fb1, fw2, fb2)


def kernel(x, edge_index, edge_attr, params):
    p = params
    f32 = jnp.float32

    x_pad = jnp.zeros((NP, DIN), f32).at[:N].set(x)
    src = jnp.zeros((EP,), jnp.int32).at[:E].set(edge_index[0])
    dst = jnp.full((EP,), N, jnp.int32).at[:E].set(edge_index[1])
    src_r = src.reshape(NW, NCH, CH)
    dst_r = dst.reshape(NW, NCH, CH)
    ea_pad = jnp.zeros((EP, ATTR), f32).at[:E].set(edge_attr)
    zeros_rows = jnp.zeros((ROWS_PER_SUB, H), f32)

    ii = jnp.arange(DIN * H)
    r_mat = (ii[None, :] // H == jnp.arange(DIN)[:, None]).astype(f32)
    g_mat = (ii[:, None] % H == jnp.arange(H)[None, :]).astype(f32)

    def row(b):
        return b.reshape(1, -1)

    w1r = p['fnn_w1'].reshape(H, 3, FH)
    fa, fb, fc = w1r[:, 0, :], w1r[:, 1, :], w1r[:, 2, :]

    xs = _sc_gather(x_pad, src_r)
    msg = _tc_edge_mlp(ea_pad, xs,
                       p['e_w1'], row(p['e_b1']),
                       p['e_w2'], row(p['e_b2']),
                       p['e_w3'], row(p['e_b3']),
                       r_mat, g_mat)
    aggp = _sc_scatter(msg, dst_r, zeros_rows)
    xc0, y0 = _tc_combine(x_pad, aggp, p['root_w'], row(p['root_b']))
    nbr1p = _sc_gather_scatter(y0, src_r, dst_r, zeros_rows)
    xc1, y1 = _tc_gin(y0, nbr1p,
                      p['gin1_w1'], row(p['gin1_b1']),
                      p['gin1_w2'], row(p['gin1_b2']),
                      p['gin1_w3'], row(p['gin1_b3']))
    nbr2p = _sc_gather_scatter(y1, src_r, dst_r, zeros_rows)
    out = _tc_gin_fnn(y1, nbr2p,
                      p['gin2_w1'], row(p['gin2_b1']),
                      p['gin2_w2'], row(p['gin2_b2']),
                      p['gin2_w3'], row(p['gin2_b3']),
                      xc0, xc1, fa, fb, fc,
                      row(p['fnn_b1']), p['fnn_w2'], row(p['fnn_b2']))
    return out[:N]


# speedup vs baseline: 3.4885x; 1.1622x over previous
import functools

import jax
import jax.numpy as jnp
from jax import lax
from jax.experimental import pallas as pl
from jax.experimental.pallas import tpu as pltpu
from jax.experimental.pallas import tpu_sc as plsc

N = 10000
E = 80000
DIN = 32
H = 32
ATTR = 4
NODE_DIM = 32
FH = 64

NP = 10000
NC = 2
NS = 16
NW = NC * NS
CH = 125
EPW = 2500
NCH = EPW // CH
EP = NW * EPW
ROWS_PER_SUB = NP // NS


def _elu(v):
    return jnp.where(v > 0, v, jnp.exp(v) - 1.0)


def _mesh():
    return plsc.VectorSubcoreMesh(
        core_axis_name="c", subcore_axis_name="s", num_cores=NC, num_subcores=NS
    )


@functools.lru_cache(maxsize=None)
def _build_sc_gather():
    @functools.partial(
        pl.kernel,
        out_type=jax.ShapeDtypeStruct((EP, H), jnp.float32),
        mesh=_mesh(),
        compiler_params=pltpu.CompilerParams(use_tc_tiling_on_sc=False),
        scratch_types=[
            pltpu.VMEM((NCH, CH), jnp.int32),
            pltpu.VMEM((EPW, H), jnp.float32),
            pltpu.SemaphoreType.DMA,
        ],
    )
    def k(table_hbm, src_hbm, out_hbm, idx_v, rows_v, sem):
        cid = lax.axis_index("c")
        sid = lax.axis_index("s")
        wid = sid * NC + cid
        pltpu.sync_copy(src_hbm.at[wid], idx_v)
        cps = [
            pltpu.async_copy(
                table_hbm.at[idx_v.at[j]], rows_v.at[pl.ds(j * CH, CH)], sem
            )
            for j in range(NCH)
        ]
        for c in cps:
            c.wait()
        pltpu.sync_copy(rows_v, out_hbm.at[pl.ds(wid * EPW, EPW)])

    return k


def _sc_gather(table, src_r):
    return _build_sc_gather()(table, src_r)


@functools.lru_cache(maxsize=None)
def _build_sc_scatter():
    @functools.partial(
        pl.kernel,
        out_type=jax.ShapeDtypeStruct((NC, NP, H), jnp.float32),
        mesh=_mesh(),
        compiler_params=pltpu.CompilerParams(use_tc_tiling_on_sc=False),
        scratch_types=[
            pltpu.VMEM((NCH, CH), jnp.int32),
            pltpu.VMEM((EPW, H), jnp.float32),
            pltpu.VMEM_SHARED((NP, H), jnp.float32),
            pltpu.SemaphoreType.DMA,
        ],
    )
    def k(vals_hbm, dst_hbm, zeros_hbm, out_hbm, idx_v, rows_v, accum, sem):
        cid = lax.axis_index("c")
        sid = lax.axis_index("s")
        wid = sid * NC + cid
        pltpu.sync_copy(zeros_hbm, accum.at[pl.ds(sid * ROWS_PER_SUB, ROWS_PER_SUB)])
        pltpu.sync_copy(dst_hbm.at[wid], idx_v)
        pltpu.sync_copy(vals_hbm.at[pl.ds(wid * EPW, EPW)], rows_v)
        plsc.subcore_barrier()
        cps = [
            pltpu.async_copy(
                rows_v.at[pl.ds(j * CH, CH)], accum.at[idx_v.at[j]], sem, add=True
            )
            for j in range(NCH)
        ]
        for c in cps:
            c.wait()
        plsc.subcore_barrier()
        sl = pl.ds(sid * ROWS_PER_SUB, ROWS_PER_SUB)
        pltpu.sync_copy(accum.at[sl], out_hbm.at[cid].at[sl])

    return k


def _sc_scatter(vals, dst_r, zeros_rows):
    return _build_sc_scatter()(vals, dst_r, zeros_rows)


@functools.lru_cache(maxsize=None)
def _build_sc_gather_scatter():
    @functools.partial(
        pl.kernel,
        out_type=jax.ShapeDtypeStruct((NC, NP, H), jnp.float32),
        mesh=_mesh(),
        compiler_params=pltpu.CompilerParams(use_tc_tiling_on_sc=False),
        scratch_types=[
            pltpu.VMEM((NCH, CH), jnp.int32),
            pltpu.VMEM((NCH, CH), jnp.int32),
            pltpu.VMEM((EPW, H), jnp.float32),
            pltpu.VMEM_SHARED((NP, H), jnp.float32),
            pltpu.SemaphoreType.DMA,
            pltpu.SemaphoreType.DMA,
        ],
    )
    def k(y_hbm, src_hbm, dst_hbm, zeros_hbm, out_hbm,
          sidx_v, didx_v, rows_v, accum, gsem, ssem):
        cid = lax.axis_index("c")
        sid = lax.axis_index("s")
        wid = sid * NC + cid
        pltpu.sync_copy(zeros_hbm, accum.at[pl.ds(sid * ROWS_PER_SUB, ROWS_PER_SUB)])
        pltpu.sync_copy(src_hbm.at[wid], sidx_v)
        pltpu.sync_copy(dst_hbm.at[wid], didx_v)
        gcps = [
            pltpu.async_copy(
                y_hbm.at[sidx_v.at[j]], rows_v.at[pl.ds(j * CH, CH)], gsem
            )
            for j in range(NCH)
        ]
        plsc.subcore_barrier()
        for c in gcps:
            c.wait()
        scps = [
            pltpu.async_copy(
                rows_v.at[pl.ds(j * CH, CH)], accum.at[didx_v.at[j]], ssem, add=True
            )
            for j in range(NCH)
        ]
        for c in scps:
            c.wait()
        plsc.subcore_barrier()
        sl = pl.ds(sid * ROWS_PER_SUB, ROWS_PER_SUB)
        pltpu.sync_copy(accum.at[sl], out_hbm.at[cid].at[sl])

    return k


def _sc_gather_scatter(y, src_r, dst_r, zeros_rows):
    return _build_sc_gather_scatter()(y, src_r, dst_r, zeros_rows)


BE = 2000


def _dot(a, b):
    return jnp.dot(a, b, preferred_element_type=jnp.float32)


def _edge_body(ea_ref, xs_ref, w1, b1, w2, b2, w3, b3, r, g, out_ref):
    h = _elu(_dot(ea_ref[...], w1[...]) + b1[...])
    h = _elu(_dot(h, w2[...]) + b2[...])
    h = _elu(_dot(h, w3[...]) + b3[...])
    xr = _dot(xs_ref[...], r[...])
    out_ref[...] = _dot(xr * h, g[...])


def _tc_edge_mlp(ea, xs, w1, b1, w2, b2, w3, b3, r, g):
    grid = (EP // BE,)

    def full(shape):
        return pl.BlockSpec(shape, lambda i: (0, 0))

    return pl.pallas_call(
        _edge_body,
        grid=grid,
        in_specs=[
            pl.BlockSpec((BE, ATTR), lambda i: (i, 0)),
            pl.BlockSpec((BE, DIN), lambda i: (i, 0)),
            full((ATTR, 256)), full((1, 256)),
            full((256, 1024)), full((1, 1024)),
            full((1024, DIN * H)), full((1, DIN * H)),
            full((DIN, DIN * H)), full((DIN * H, H)),
        ],
        out_specs=pl.BlockSpec((BE, H), lambda i: (i, 0)),
        out_shape=jax.ShapeDtypeStruct((EP, H), jnp.float32),
    )(ea, xs, w1, b1, w2, b2, w3, b3, r, g)


BN = 2000


def _combine_body(x_ref, aggp_ref, rw, rb, xc_ref, y_ref):
    agg = aggp_ref[0] + aggp_ref[1]
    xc = _dot(x_ref[...], rw[...]) + agg + rb[...]
    xc_ref[...] = xc
    y_ref[...] = _elu(xc)


def _tc_combine(x, aggp, rw, rb):
    grid = (NP // BN,)
    return pl.pallas_call(
        _combine_body,
        grid=grid,
        in_specs=[
            pl.BlockSpec((BN, DIN), lambda i: (i, 0)),
            pl.BlockSpec((NC, BN, H), lambda i: (0, i, 0)),
            pl.BlockSpec((DIN, H), lambda i: (0, 0)),
            pl.BlockSpec((1, H), lambda i: (0, 0)),
        ],
        out_specs=[
            pl.BlockSpec((BN, H), lambda i: (i, 0)),
            pl.BlockSpec((BN, H), lambda i: (i, 0)),
        ],
        out_shape=[
            jax.ShapeDtypeStruct((NP, H), jnp.float32),
            jax.ShapeDtypeStruct((NP, H), jnp.float32),
        ],
    )(x, aggp, rw, rb)


def _gin_body(y_ref, nbrp_ref, w1, b1, w2, b2, w3, b3, xc_ref, yout_ref):
    t = y_ref[...] + nbrp_ref[0] + nbrp_ref[1]
    h = _elu(_dot(t, w1[...]) + b1[...])
    h = _elu(_dot(h, w2[...]) + b2[...])
    xc = _dot(h, w3[...]) + b3[...]
    xc_ref[...] = xc
    yout_ref[...] = _elu(xc)


def _tc_gin(y, nbrp, w1, b1, w2, b2, w3, b3):
    grid = (NP // BN,)

    def wspec(shape):
        return pl.BlockSpec(shape, lambda i: (0, 0))

    return pl.pallas_call(
        _gin_body,
        grid=grid,
        in_specs=[
            pl.BlockSpec((BN, H), lambda i: (i, 0)),
            pl.BlockSpec((NC, BN, H), lambda i: (0, i, 0)),
            wspec((H, H)), wspec((1, H)),
            wspec((H, H)), wspec((1, H)),
            wspec((H, H)), wspec((1, H)),
        ],
        out_specs=[
            pl.BlockSpec((BN, H), lambda i: (i, 0)),
            pl.BlockSpec((BN, H), lambda i: (i, 0)),
        ],
        out_shape=[
            jax.ShapeDtypeStruct((NP, H), jnp.float32),
            jax.ShapeDtypeStruct((NP, H), jnp.float32),
        ],
    )(y, nbrp, w1, b1, w2, b2, w3, b3)


def _gin_fnn_body(y_ref, nbrp_ref, w1, b1, w2, b2, w3, b3,
                  xc0_ref, xc1_ref, fa, fb, fc, fb1, fw2, fb2, out_ref):
    t = y_ref[...] + nbrp_ref[0] + nbrp_ref[1]
    h = _elu(_dot(t, w1[...]) + b1[...])
    h = _elu(_dot(h, w2[...]) + b2[...])
    xc2 = _dot(h, w3[...]) + b3[...]
    hh = (_dot(xc0_ref[...], fa[...]) + _dot(xc1_ref[...], fb[...])
          + _dot(xc2, fc[...]) + fb1[...])
    hh = _elu(hh)
    out_ref[...] = _dot(hh, fw2[...]) + fb2[...]


def _tc_gin_fnn(y, nbrp, w1, b1, w2, b2, w3, b3, xc0, xc1, fa, fb, fc, fb1, fw2, fb2):
    grid = (NP // BN,)

    def wspec(shape):
        return pl.BlockSpec(shape, lambda i: (0, 0))

    def nspec():
        return pl.BlockSpec((BN, H), lambda i: (i, 0))

    return pl.pallas_call(
        _gin_fnn_body,
        grid=grid,
        in_specs=[
            nspec(),
            pl.BlockSpec((NC, BN, H), lambda i: (0, i, 0)),
            wspec((H, H)), wspec((1, H)),
            wspec((H, H)), wspec((1, H)),
            wspec((H, H)), wspec((1, H)),
            nspec(), nspec(),
            wspec((H, FH)), wspec((H, FH)), wspec((H, FH)), wspec((1, FH)),
            wspec((FH, NODE_DIM)), wspec((1, NODE_DIM)),
        ],
        out_specs=pl.BlockSpec((BN, NODE_DIM), lambda i: (i, 0)),
        out_shape=jax.ShapeDtypeStruct((NP, NODE_DIM), jnp.float32),
    )(y, nbrp, w1, b1, w2, b2, w3, b3, xc0, xc1, fa, fb, fc, fb1, fw2, fb2)


def kernel(x, edge_index, edge_attr, params):
    p = params
    f32 = jnp.float32

    src_r = edge_index[0].reshape(NW, NCH, CH)
    dst_r = edge_index[1].reshape(NW, NCH, CH)
    zeros_rows = jnp.zeros((ROWS_PER_SUB, H), f32)

    ii = jnp.arange(DIN * H)
    r_mat = (ii[None, :] // H == jnp.arange(DIN)[:, None]).astype(f32)
    g_mat = (ii[:, None] % H == jnp.arange(H)[None, :]).astype(f32)

    def row(b):
        return b.reshape(1, -1)

    w1r = p['fnn_w1'].reshape(H, 3, FH)
    fa, fb, fc = w1r[:, 0, :], w1r[:, 1, :], w1r[:, 2, :]

    xs = _sc_gather(x, src_r)
    msg = _tc_edge_mlp(edge_attr, xs,
                       p['e_w1'], row(p['e_b1']),
                       p['e_w2'], row(p['e_b2']),
                       p['e_w3'], row(p['e_b3']),
                       r_mat, g_mat)
    aggp = _sc_scatter(msg, dst_r, zeros_rows)
    xc0, y0 = _tc_combine(x, aggp, p['root_w'], row(p['root_b']))
    nbr1p = _sc_gather_scatter(y0, src_r, dst_r, zeros_rows)
    xc1, y1 = _tc_gin(y0, nbr1p,
                      p['gin1_w1'], row(p['gin1_b1']),
                      p['gin1_w2'], row(p['gin1_b2']),
                      p['gin1_w3'], row(p['gin1_b3']))
    nbr2p = _sc_gather_scatter(y1, src_r, dst_r, zeros_rows)
    out = _tc_gin_fnn(y1, nbr2p,
                      p['gin2_w1'], row(p['gin2_b1']),
                      p['gin2_w2'], row(p['gin2_b2']),
                      p['gin2_w3'], row(p['gin2_b3']),
                      xc0, xc1, fa, fb, fc,
                      row(p['fnn_b1']), p['fnn_w2'], row(p['fnn_b2']))
    return out


# speedup vs baseline: 3.7339x; 1.0704x over previous
import functools

import jax
import jax.numpy as jnp
from jax import lax
from jax.experimental import pallas as pl
from jax.experimental.pallas import tpu as pltpu
from jax.experimental.pallas import tpu_sc as plsc

N = 10000
E = 80000
DIN = 32
H = 32
ATTR = 4
NODE_DIM = 32
FH = 64

W = 128
NC = 2
NS = 16
NW = NC * NS
CH = 125
EPW = 2500
NCH = EPW // CH
ROWS_PER_SUB = N // NS
RING = 5
ROUNDS = NCH // RING


def _elu(v):
    return jnp.where(v > 0, v, jnp.exp(v) - 1.0)


def _mesh():
    return plsc.VectorSubcoreMesh(
        core_axis_name="c", subcore_axis_name="s", num_cores=NC, num_subcores=NS
    )


@functools.lru_cache(maxsize=None)
def _build_sc_gather():
    @functools.partial(
        pl.kernel,
        out_type=jax.ShapeDtypeStruct((E, W), jnp.float32),
        mesh=_mesh(),
        compiler_params=pltpu.CompilerParams(use_tc_tiling_on_sc=False),
        scratch_types=[
            pltpu.VMEM((NCH, CH), jnp.int32),
            pltpu.VMEM((RING * CH, W), jnp.float32),
            pltpu.SemaphoreType.DMA,
            pltpu.SemaphoreType.DMA,
        ],
    )
    def k(table_hbm, src_hbm, out_hbm, idx_v, slots_v, gsem, osem):
        cid = lax.axis_index("c")
        sid = lax.axis_index("s")
        wid = sid * NC + cid
        pltpu.sync_copy(src_hbm.at[wid], idx_v)
        base = wid * EPW
        ocps = []
        for r in range(ROUNDS):
            for c in ocps:
                c.wait()
            gcps = []
            for t in range(RING):
                j = r * RING + t
                slot = slots_v.at[pl.ds(t * CH, CH)]
                gcps.append(
                    pltpu.async_copy(table_hbm.at[idx_v.at[j]], slot, gsem)
                )
            for c in gcps:
                c.wait()
            ocps = []
            for t in range(RING):
                j = r * RING + t
                slot = slots_v.at[pl.ds(t * CH, CH)]
                ocps.append(
                    pltpu.async_copy(
                        slot, out_hbm.at[pl.ds(base + j * CH, CH)], osem
                    )
                )
        for c in ocps:
            c.wait()

    return k


def _sc_gather(table, src_r):
    return _build_sc_gather()(table, src_r)


@functools.lru_cache(maxsize=None)
def _build_sc_scatter():
    @functools.partial(
        pl.kernel,
        out_type=jax.ShapeDtypeStruct((NC, N, W), jnp.float32),
        mesh=_mesh(),
        compiler_params=pltpu.CompilerParams(use_tc_tiling_on_sc=False),
        scratch_types=[
            pltpu.VMEM((NCH, CH), jnp.int32),
            pltpu.VMEM((RING * CH, H), jnp.float32),
            pltpu.VMEM_SHARED((N, H), jnp.float32),
            pltpu.SemaphoreType.DMA,
            pltpu.SemaphoreType.DMA,
        ],
    )
    def k(vals_hbm, dst_hbm, zeros_hbm, out_hbm, idx_v, slots_v, accum,
          lsem, ssem):
        cid = lax.axis_index("c")
        sid = lax.axis_index("s")
        wid = sid * NC + cid
        pltpu.sync_copy(
            zeros_hbm, accum.at[pl.ds(sid * ROWS_PER_SUB, ROWS_PER_SUB)]
        )
        pltpu.sync_copy(dst_hbm.at[wid], idx_v)
        base = wid * EPW
        plsc.subcore_barrier()
        scps = []
        for r in range(ROUNDS):
            for c in scps:
                c.wait()
            lcps = [
                pltpu.async_copy(
                    vals_hbm.at[pl.ds(base + (r * RING + t) * CH, CH),
                                pl.ds(0, H)],
                    slots_v.at[pl.ds(t * CH, CH)], lsem,
                )
                for t in range(RING)
            ]
            for c in lcps:
                c.wait()
            scps = []
            for t in range(RING):
                j = r * RING + t
                slot = slots_v.at[pl.ds(t * CH, CH)]
                scps.append(
                    pltpu.async_copy(
                        slot, accum.at[idx_v.at[j]], ssem, add=True
                    )
                )
        for c in scps:
            c.wait()
        plsc.subcore_barrier()
        sl = pl.ds(sid * ROWS_PER_SUB, ROWS_PER_SUB)
        pltpu.sync_copy(accum.at[sl], out_hbm.at[cid].at[sl, pl.ds(0, H)])

    return k


def _sc_scatter(vals, dst_r, zeros_rows):
    return _build_sc_scatter()(vals, dst_r, zeros_rows)


@functools.lru_cache(maxsize=None)
def _build_sc_gather_scatter():
    @functools.partial(
        pl.kernel,
        out_type=jax.ShapeDtypeStruct((NC, N, H), jnp.float32),
        mesh=_mesh(),
        compiler_params=pltpu.CompilerParams(use_tc_tiling_on_sc=False),
        scratch_types=[
            pltpu.VMEM((NCH, CH), jnp.int32),
            pltpu.VMEM((NCH, CH), jnp.int32),
            pltpu.VMEM((EPW, H), jnp.float32),
            pltpu.VMEM_SHARED((N, H), jnp.float32),
            pltpu.SemaphoreType.DMA,
            pltpu.SemaphoreType.DMA,
        ],
    )
    def k(y_hbm, src_hbm, dst_hbm, zeros_hbm, out_hbm,
          sidx_v, didx_v, rows_v, accum, gsem, ssem):
        cid = lax.axis_index("c")
        sid = lax.axis_index("s")
        wid = sid * NC + cid
        pltpu.sync_copy(
            zeros_hbm, accum.at[pl.ds(sid * ROWS_PER_SUB, ROWS_PER_SUB)]
        )
        pltpu.sync_copy(src_hbm.at[wid], sidx_v)
        pltpu.sync_copy(dst_hbm.at[wid], didx_v)
        gcps = [
            pltpu.async_copy(
                y_hbm.at[sidx_v.at[j]], rows_v.at[pl.ds(j * CH, CH)], gsem
            )
            for j in range(NCH)
        ]
        plsc.subcore_barrier()
        for c in gcps:
            c.wait()
        scps = [
            pltpu.async_copy(
                rows_v.at[pl.ds(j * CH, CH)], accum.at[didx_v.at[j]], ssem,
                add=True,
            )
            for j in range(NCH)
        ]
        for c in scps:
            c.wait()
        plsc.subcore_barrier()
        sl = pl.ds(sid * ROWS_PER_SUB, ROWS_PER_SUB)
        pltpu.sync_copy(accum.at[sl], out_hbm.at[cid].at[sl])

    return k


def _sc_gather_scatter(y, src_r, dst_r, zeros_rows):
    return _build_sc_gather_scatter()(y, src_r, dst_r, zeros_rows)


BE = 2000


def _dot(a, b):
    return jnp.dot(a, b, preferred_element_type=jnp.float32)


def _wide(v, rows):
    return jnp.concatenate([v, jnp.zeros((rows, W - H), jnp.float32)], axis=1)


def _edge_body(ea_ref, xs_ref, w1, b1, w2, b2, w3, b3, r, g, out_ref):
    h = _elu(_dot(ea_ref[...], w1[...]) + b1[...])
    h = _elu(_dot(h, w2[...]) + b2[...])
    h = _elu(_dot(h, w3[...]) + b3[...])
    xr = _dot(xs_ref[:, :DIN], r[...])
    out_ref[...] = _wide(_dot(xr * h, g[...]), BE)


def _tc_edge_mlp(ea, xs, w1, b1, w2, b2, w3, b3, r, g):
    grid = (E // BE,)

    def full(shape):
        return pl.BlockSpec(shape, lambda i: (0, 0))

    return pl.pallas_call(
        _edge_body,
        grid=grid,
        in_specs=[
            pl.BlockSpec((BE, ATTR), lambda i: (i, 0)),
            pl.BlockSpec((BE, W), lambda i: (i, 0)),
            full((ATTR, 256)), full((1, 256)),
            full((256, 1024)), full((1, 1024)),
            full((1024, DIN * H)), full((1, DIN * H)),
            full((DIN, DIN * H)), full((DIN * H, H)),
        ],
        out_specs=pl.BlockSpec((BE, W), lambda i: (i, 0)),
        out_shape=jax.ShapeDtypeStruct((E, W), jnp.float32),
    )(ea, xs, w1, b1, w2, b2, w3, b3, r, g)


BN = 2000


def _combine_body(x_ref, aggp_ref, rw, rb, xc_ref, y_ref):
    agg = aggp_ref[0][:, :H] + aggp_ref[1][:, :H]
    xc = _dot(x_ref[:, :DIN], rw[...]) + agg + rb[...]
    xc_ref[...] = xc
    y_ref[...] = _elu(xc)


def _tc_combine(x, aggp, rw, rb):
    grid = (N // BN,)
    return pl.pallas_call(
        _combine_body,
        grid=grid,
        in_specs=[
            pl.BlockSpec((BN, W), lambda i: (i, 0)),
            pl.BlockSpec((NC, BN, W), lambda i: (0, i, 0)),
            pl.BlockSpec((DIN, H), lambda i: (0, 0)),
            pl.BlockSpec((1, H), lambda i: (0, 0)),
        ],
        out_specs=[
            pl.BlockSpec((BN, H), lambda i: (i, 0)),
            pl.BlockSpec((BN, H), lambda i: (i, 0)),
        ],
        out_shape=[
            jax.ShapeDtypeStruct((N, H), jnp.float32),
            jax.ShapeDtypeStruct((N, H), jnp.float32),
        ],
    )(x, aggp, rw, rb)


def _gin_body(y_ref, nbrp_ref, w1, b1, w2, b2, w3, b3, xc_ref, yout_ref):
    t = y_ref[...] + nbrp_ref[0] + nbrp_ref[1]
    h = _elu(_dot(t, w1[...]) + b1[...])
    h = _elu(_dot(h, w2[...]) + b2[...])
    xc = _dot(h, w3[...]) + b3[...]
    xc_ref[...] = xc
    yout_ref[...] = _elu(xc)


def _tc_gin(y, nbrp, w1, b1, w2, b2, w3, b3):
    grid = (N // BN,)

    def wspec(shape):
        return pl.BlockSpec(shape, lambda i: (0, 0))

    return pl.pallas_call(
        _gin_body,
        grid=grid,
        in_specs=[
            pl.BlockSpec((BN, H), lambda i: (i, 0)),
            pl.BlockSpec((NC, BN, H), lambda i: (0, i, 0)),
            wspec((H, H)), wspec((1, H)),
            wspec((H, H)), wspec((1, H)),
            wspec((H, H)), wspec((1, H)),
        ],
        out_specs=[
            pl.BlockSpec((BN, H), lambda i: (i, 0)),
            pl.BlockSpec((BN, H), lambda i: (i, 0)),
        ],
        out_shape=[
            jax.ShapeDtypeStruct((N, H), jnp.float32),
            jax.ShapeDtypeStruct((N, H), jnp.float32),
        ],
    )(y, nbrp, w1, b1, w2, b2, w3, b3)


def _gin_fnn_body(y_ref, nbrp_ref, w1, b1, w2, b2, w3, b3,
                  xc0_ref, xc1_ref, fa, fb, fc, fb1, fw2, fb2, out_ref):
    t = y_ref[...] + nbrp_ref[0] + nbrp_ref[1]
    h = _elu(_dot(t, w1[...]) + b1[...])
    h = _elu(_dot(h, w2[...]) + b2[...])
    xc2 = _dot(h, w3[...]) + b3[...]
    hh = (_dot(xc0_ref[...], fa[...]) + _dot(xc1_ref[...], fb[...])
          + _dot(xc2, fc[...]) + fb1[...])
    hh = _elu(hh)
    out_ref[...] = _dot(hh, fw2[...]) + fb2[...]


def _tc_gin_fnn(y, nbrp, w1, b1, w2, b2, w3, b3, xc0, xc1, fa, fb, fc, fb1, fw2, fb2):
    grid = (N // BN,)

    def wspec(shape):
        return pl.BlockSpec(shape, lambda i: (0, 0))

    def nspec():
        return pl.BlockSpec((BN, H), lambda i: (i, 0))

    return pl.pallas_call(
        _gin_fnn_body,
        grid=grid,
        in_specs=[
            pl.BlockSpec((BN, H), lambda i: (i, 0)),
            pl.BlockSpec((NC, BN, H), lambda i: (0, i, 0)),
            wspec((H, H)), wspec((1, H)),
            wspec((H, H)), wspec((1, H)),
            wspec((H, H)), wspec((1, H)),
            nspec(), nspec(),
            wspec((H, FH)), wspec((H, FH)), wspec((H, FH)), wspec((1, FH)),
            wspec((FH, NODE_DIM)), wspec((1, NODE_DIM)),
        ],
        out_specs=pl.BlockSpec((BN, NODE_DIM), lambda i: (i, 0)),
        out_shape=jax.ShapeDtypeStruct((N, NODE_DIM), jnp.float32),
    )(y, nbrp, w1, b1, w2, b2, w3, b3, xc0, xc1, fa, fb, fc, fb1, fw2, fb2)


def kernel(x, edge_index, edge_attr, params):
    p = params
    f32 = jnp.float32

    xw = jnp.pad(x, ((0, 0), (0, W - DIN)))
    src_r = edge_index[0].reshape(NW, NCH, CH)
    dst_r = edge_index[1].reshape(NW, NCH, CH)
    zeros_rows = jnp.zeros((ROWS_PER_SUB, H), f32)

    ii = jnp.arange(DIN * H)
    r_mat = (ii[None, :] // H == jnp.arange(DIN)[:, None]).astype(f32)
    g_mat = (ii[:, None] % H == jnp.arange(H)[None, :]).astype(f32)

    def row(b):
        return b.reshape(1, -1)

    w1r = p['fnn_w1'].reshape(H, 3, FH)
    fa, fb, fc = w1r[:, 0, :], w1r[:, 1, :], w1r[:, 2, :]

    xs = _sc_gather(xw, src_r)
    msg = _tc_edge_mlp(edge_attr, xs,
                       p['e_w1'], row(p['e_b1']),
                       p['e_w2'], row(p['e_b2']),
                       p['e_w3'], row(p['e_b3']),
                       r_mat, g_mat)
    aggp = _sc_scatter(msg, dst_r, zeros_rows)
    xc0, y0 = _tc_combine(xw, aggp, p['root_w'], row(p['root_b']))
    nbr1p = _sc_gather_scatter(y0, src_r, dst_r, zeros_rows)
    xc1, y1 = _tc_gin(y0, nbr1p,
                      p['gin1_w1'], row(p['gin1_b1']),
                      p['gin1_w2'], row(p['gin1_b2']),
                      p['gin1_w3'], row(p['gin1_b3']))
    nbr2p = _sc_gather_scatter(y1, src_r, dst_r, zeros_rows)
    out = _tc_gin_fnn(y1, nbr2p,
                      p['gin2_w1'], row(p['gin2_b1']),
                      p['gin2_w2'], row(p['gin2_b2']),
                      p['gin2_w3'], row(p['gin2_b3']),
                      xc0, xc1, fa, fb, fc,
                      row(p['fnn_b1']), p['fnn_w2'], row(p['fnn_b2']))
    return out


# speedup vs baseline: 3.8221x; 1.0236x over previous
import functools

import jax
import jax.numpy as jnp
from jax import lax
from jax.experimental import pallas as pl
from jax.experimental.pallas import tpu as pltpu
from jax.experimental.pallas import tpu_sc as plsc

N = 10000
E = 80000
DIN = 32
H = 32
ATTR = 4
NODE_DIM = 32
FH = 64

W = 128
NC = 2
NS = 16
NW = NC * NS
CH = 125
EPW = 2500
NCH = EPW // CH
ROWS_PER_SUB = N // NS
RING = 5
ROUNDS = NCH // RING


def _elu(v):
    return jnp.where(v > 0, v, jnp.exp(v) - 1.0)


def _mesh():
    return plsc.VectorSubcoreMesh(
        core_axis_name="c", subcore_axis_name="s", num_cores=NC, num_subcores=NS
    )


@functools.lru_cache(maxsize=None)
def _build_sc_gather():
    @functools.partial(
        pl.kernel,
        out_type=jax.ShapeDtypeStruct((E, W), jnp.float32),
        mesh=_mesh(),
        compiler_params=pltpu.CompilerParams(use_tc_tiling_on_sc=False),
        scratch_types=[
            pltpu.VMEM((NCH, CH), jnp.int32),
            pltpu.VMEM((RING * CH, W), jnp.float32),
            pltpu.SemaphoreType.DMA,
            pltpu.SemaphoreType.DMA,
        ],
    )
    def k(table_hbm, src_hbm, out_hbm, idx_v, slots_v, gsem, osem):
        cid = lax.axis_index("c")
        sid = lax.axis_index("s")
        wid = sid * NC + cid
        pltpu.sync_copy(src_hbm.at[wid], idx_v)
        base = wid * EPW
        ocps = []
        for r in range(ROUNDS):
            for c in ocps:
                c.wait()
            gcps = []
            for t in range(RING):
                j = r * RING + t
                slot = slots_v.at[pl.ds(t * CH, CH)]
                gcps.append(
                    pltpu.async_copy(table_hbm.at[idx_v.at[j]], slot, gsem)
                )
            for c in gcps:
                c.wait()
            ocps = []
            for t in range(RING):
                j = r * RING + t
                slot = slots_v.at[pl.ds(t * CH, CH)]
                ocps.append(
                    pltpu.async_copy(
                        slot, out_hbm.at[pl.ds(base + j * CH, CH)], osem
                    )
                )
        for c in ocps:
            c.wait()

    return k


def _sc_gather(table, src_r):
    return _build_sc_gather()(table, src_r)


@functools.lru_cache(maxsize=None)
def _build_sc_scatter():
    @functools.partial(
        pl.kernel,
        out_type=jax.ShapeDtypeStruct((NC, N, W), jnp.float32),
        mesh=_mesh(),
        compiler_params=pltpu.CompilerParams(use_tc_tiling_on_sc=False),
        scratch_types=[
            pltpu.VMEM((NCH, CH), jnp.int32),
            pltpu.VMEM((RING * CH, H), jnp.float32),
            pltpu.VMEM_SHARED((N, H), jnp.float32),
            pltpu.SemaphoreType.DMA,
            pltpu.SemaphoreType.DMA,
        ],
    )
    def k(vals_hbm, dst_hbm, zeros_hbm, out_hbm, idx_v, slots_v, accum,
          lsem, ssem):
        cid = lax.axis_index("c")
        sid = lax.axis_index("s")
        wid = sid * NC + cid
        pltpu.sync_copy(
            zeros_hbm, accum.at[pl.ds(sid * ROWS_PER_SUB, ROWS_PER_SUB)]
        )
        pltpu.sync_copy(dst_hbm.at[wid], idx_v)
        base = wid * EPW
        plsc.subcore_barrier()
        scps = []
        for r in range(ROUNDS):
            for c in scps:
                c.wait()
            lcps = [
                pltpu.async_copy(
                    vals_hbm.at[pl.ds(base + (r * RING + t) * CH, CH),
                                pl.ds(0, H)],
                    slots_v.at[pl.ds(t * CH, CH)], lsem,
                )
                for t in range(RING)
            ]
            for c in lcps:
                c.wait()
            scps = []
            for t in range(RING):
                j = r * RING + t
                slot = slots_v.at[pl.ds(t * CH, CH)]
                scps.append(
                    pltpu.async_copy(
                        slot, accum.at[idx_v.at[j]], ssem, add=True
                    )
                )
        for c in scps:
            c.wait()
        plsc.subcore_barrier()
        sl = pl.ds(sid * ROWS_PER_SUB, ROWS_PER_SUB)
        pltpu.sync_copy(accum.at[sl], out_hbm.at[cid].at[sl, pl.ds(0, H)])

    return k


def _sc_scatter(vals, dst_r, zeros_rows):
    return _build_sc_scatter()(vals, dst_r, zeros_rows)


@functools.lru_cache(maxsize=None)
def _build_sc_gather_scatter():
    @functools.partial(
        pl.kernel,
        out_type=jax.ShapeDtypeStruct((NC, N, W), jnp.float32),
        mesh=_mesh(),
        compiler_params=pltpu.CompilerParams(use_tc_tiling_on_sc=False),
        scratch_types=[
            pltpu.VMEM((NCH, CH), jnp.int32),
            pltpu.VMEM((NCH, CH), jnp.int32),
            pltpu.VMEM((EPW, H), jnp.float32),
            pltpu.VMEM_SHARED((N, H), jnp.float32),
            pltpu.SemaphoreType.DMA,
            pltpu.SemaphoreType.DMA,
        ],
    )
    def k(y_hbm, src_hbm, dst_hbm, zeros_hbm, out_hbm,
          sidx_v, didx_v, rows_v, accum, gsem, ssem):
        cid = lax.axis_index("c")
        sid = lax.axis_index("s")
        wid = sid * NC + cid
        pltpu.sync_copy(
            zeros_hbm, accum.at[pl.ds(sid * ROWS_PER_SUB, ROWS_PER_SUB)]
        )
        pltpu.sync_copy(src_hbm.at[wid], sidx_v)
        pltpu.sync_copy(dst_hbm.at[wid], didx_v)
        gcps = [
            pltpu.async_copy(
                y_hbm.at[sidx_v.at[j]], rows_v.at[pl.ds(j * CH, CH)], gsem
            )
            for j in range(NCH)
        ]
        plsc.subcore_barrier()
        for c in gcps:
            c.wait()
        scps = [
            pltpu.async_copy(
                rows_v.at[pl.ds(j * CH, CH)], accum.at[didx_v.at[j]], ssem,
                add=True,
            )
            for j in range(NCH)
        ]
        for c in scps:
            c.wait()
        plsc.subcore_barrier()
        sl = pl.ds(sid * ROWS_PER_SUB, ROWS_PER_SUB)
        pltpu.sync_copy(accum.at[sl], out_hbm.at[cid].at[sl, pl.ds(0, H)])

    return k


def _sc_gather_scatter(y, src_r, dst_r, zeros_rows):
    return _build_sc_gather_scatter()(y, src_r, dst_r, zeros_rows)


BE = 2000


def _dot(a, b):
    return jnp.dot(a, b, preferred_element_type=jnp.float32)


def _wide(v, rows):
    return jnp.concatenate([v, jnp.zeros((rows, W - H), jnp.float32)], axis=1)


def _edge_body(ea_ref, xs_ref, w1, b1, w2, b2, w3, b3, r, g, out_ref):
    h = _elu(_dot(ea_ref[...], w1[...]) + b1[...])
    h = _elu(_dot(h, w2[...]) + b2[...])
    h = _elu(_dot(h, w3[...]) + b3[...])
    xr = _dot(xs_ref[:, :DIN], r[...])
    out_ref[...] = _wide(_dot(xr * h, g[...]), BE)


def _tc_edge_mlp(ea, xs, w1, b1, w2, b2, w3, b3, r, g):
    grid = (E // BE,)

    def full(shape):
        return pl.BlockSpec(shape, lambda i: (0, 0))

    return pl.pallas_call(
        _edge_body,
        grid=grid,
        in_specs=[
            pl.BlockSpec((BE, ATTR), lambda i: (i, 0)),
            pl.BlockSpec((BE, W), lambda i: (i, 0)),
            full((ATTR, 256)), full((1, 256)),
            full((256, 1024)), full((1, 1024)),
            full((1024, DIN * H)), full((1, DIN * H)),
            full((DIN, DIN * H)), full((DIN * H, H)),
        ],
        out_specs=pl.BlockSpec((BE, W), lambda i: (i, 0)),
        out_shape=jax.ShapeDtypeStruct((E, W), jnp.float32),
    )(ea, xs, w1, b1, w2, b2, w3, b3, r, g)


BN = 2000


def _combine_body(x_ref, aggp_ref, rw, rb, xc_ref, y_ref):
    agg = aggp_ref[0][:, :H] + aggp_ref[1][:, :H]
    xc = _dot(x_ref[:, :DIN], rw[...]) + agg + rb[...]
    xc_ref[...] = xc
    y_ref[...] = _elu(xc)


def _tc_combine(x, aggp, rw, rb):
    grid = (N // BN,)
    return pl.pallas_call(
        _combine_body,
        grid=grid,
        in_specs=[
            pl.BlockSpec((BN, W), lambda i: (i, 0)),
            pl.BlockSpec((NC, BN, W), lambda i: (0, i, 0)),
            pl.BlockSpec((DIN, H), lambda i: (0, 0)),
            pl.BlockSpec((1, H), lambda i: (0, 0)),
        ],
        out_specs=[
            pl.BlockSpec((BN, H), lambda i: (i, 0)),
            pl.BlockSpec((BN, H), lambda i: (i, 0)),
        ],
        out_shape=[
            jax.ShapeDtypeStruct((N, H), jnp.float32),
            jax.ShapeDtypeStruct((N, H), jnp.float32),
        ],
    )(x, aggp, rw, rb)


def _gin_body(y_ref, nbrp_ref, w1, b1, w2, b2, w3, b3, xc_ref, yout_ref):
    t = y_ref[...] + nbrp_ref[0][:, :H] + nbrp_ref[1][:, :H]
    h = _elu(_dot(t, w1[...]) + b1[...])
    h = _elu(_dot(h, w2[...]) + b2[...])
    xc = _dot(h, w3[...]) + b3[...]
    xc_ref[...] = xc
    yout_ref[...] = _elu(xc)


def _tc_gin(y, nbrp, w1, b1, w2, b2, w3, b3):
    grid = (N // BN,)

    def wspec(shape):
        return pl.BlockSpec(shape, lambda i: (0, 0))

    return pl.pallas_call(
        _gin_body,
        grid=grid,
        in_specs=[
            pl.BlockSpec((BN, H), lambda i: (i, 0)),
            pl.BlockSpec((NC, BN, W), lambda i: (0, i, 0)),
            wspec((H, H)), wspec((1, H)),
            wspec((H, H)), wspec((1, H)),
            wspec((H, H)), wspec((1, H)),
        ],
        out_specs=[
            pl.BlockSpec((BN, H), lambda i: (i, 0)),
            pl.BlockSpec((BN, H), lambda i: (i, 0)),
        ],
        out_shape=[
            jax.ShapeDtypeStruct((N, H), jnp.float32),
            jax.ShapeDtypeStruct((N, H), jnp.float32),
        ],
    )(y, nbrp, w1, b1, w2, b2, w3, b3)


def _gin_fnn_body(y_ref, nbrp_ref, w1, b1, w2, b2, w3, b3,
                  xc0_ref, xc1_ref, fa, fb, fc, fb1, fw2, fb2, out_ref):
    t = y_ref[...] + nbrp_ref[0][:, :H] + nbrp_ref[1][:, :H]
    h = _elu(_dot(t, w1[...]) + b1[...])
    h = _elu(_dot(h, w2[...]) + b2[...])
    xc2 = _dot(h, w3[...]) + b3[...]
    hh = (_dot(xc0_ref[...], fa[...]) + _dot(xc1_ref[...], fb[...])
          + _dot(xc2, fc[...]) + fb1[...])
    hh = _elu(hh)
    out_ref[...] = _dot(hh, fw2[...]) + fb2[...]


def _tc_gin_fnn(y, nbrp, w1, b1, w2, b2, w3, b3, xc0, xc1, fa, fb, fc, fb1, fw2, fb2):
    grid = (N // BN,)

    def wspec(shape):
        return pl.BlockSpec(shape, lambda i: (0, 0))

    def nspec():
        return pl.BlockSpec((BN, H), lambda i: (i, 0))

    return pl.pallas_call(
        _gin_fnn_body,
        grid=grid,
        in_specs=[
            pl.BlockSpec((BN, H), lambda i: (i, 0)),
            pl.BlockSpec((NC, BN, W), lambda i: (0, i, 0)),
            wspec((H, H)), wspec((1, H)),
            wspec((H, H)), wspec((1, H)),
            wspec((H, H)), wspec((1, H)),
            nspec(), nspec(),
            wspec((H, FH)), wspec((H, FH)), wspec((H, FH)), wspec((1, FH)),
            wspec((FH, NODE_DIM)), wspec((1, NODE_DIM)),
        ],
        out_specs=pl.BlockSpec((BN, NODE_DIM), lambda i: (i, 0)),
        out_shape=jax.ShapeDtypeStruct((N, NODE_DIM), jnp.float32),
    )(y, nbrp, w1, b1, w2, b2, w3, b3, xc0, xc1, fa, fb, fc, fb1, fw2, fb2)


def kernel(x, edge_index, edge_attr, params):
    p = params
    f32 = jnp.float32

    xw = jnp.pad(x, ((0, 0), (0, W - DIN)))
    src_r = edge_index[0].reshape(NW, NCH, CH)
    dst_r = edge_index[1].reshape(NW, NCH, CH)
    zeros_rows = jnp.zeros((ROWS_PER_SUB, H), f32)

    ii = jnp.arange(DIN * H)
    r_mat = (ii[None, :] // H == jnp.arange(DIN)[:, None]).astype(f32)
    g_mat = (ii[:, None] % H == jnp.arange(H)[None, :]).astype(f32)

    def row(b):
        return b.reshape(1, -1)

    w1r = p['fnn_w1'].reshape(H, 3, FH)
    fa, fb, fc = w1r[:, 0, :], w1r[:, 1, :], w1r[:, 2, :]

    xs = _sc_gather(xw, src_r)
    msg = _tc_edge_mlp(edge_attr, xs,
                       p['e_w1'], row(p['e_b1']),
                       p['e_w2'], row(p['e_b2']),
                       p['e_w3'], row(p['e_b3']),
                       r_mat, g_mat)
    aggp = _sc_scatter(msg, dst_r, zeros_rows)
    xc0, y0 = _tc_combine(xw, aggp, p['root_w'], row(p['root_b']))
    nbr1p = _sc_gather_scatter(y0, src_r, dst_r, zeros_rows)
    xc1, y1 = _tc_gin(y0, nbr1p,
                      p['gin1_w1'], row(p['gin1_b1']),
                      p['gin1_w2'], row(p['gin1_b2']),
                      p['gin1_w3'], row(p['gin1_b3']))
    nbr2p = _sc_gather_scatter(y1, src_r, dst_r, zeros_rows)
    out = _tc_gin_fnn(y1, nbr2p,
                      p['gin2_w1'], row(p['gin2_b1']),
                      p['gin2_w2'], row(p['gin2_b2']),
                      p['gin2_w3'], row(p['gin2_b3']),
                      xc0, xc1, fa, fb, fc,
                      row(p['fnn_b1']), p['fnn_w2'], row(p['fnn_b2']))
    return out
